# Initial kernel scaffold; baseline (speedup 1.0000x reference)
import jax
import jax.numpy as jnp
from jax.experimental import pallas as pl


def kernel(img):
    raise NotImplementedError("write your pallas kernel here")


# speedup vs baseline: 1.5680x; 1.5680x over previous
import functools

import jax
import jax.numpy as jnp
from jax import lax
from jax.experimental import pallas as pl
from jax.experimental.pallas import tpu as pltpu
from jax.experimental.pallas import tpu_sc as plsc

N, C, H, W = 32, 3, 512, 512
GRID = 4
PH, PW = H // GRID, W // GRID
ROWS_PER_SAMPLE = C * H * (W // PW)
TOTAL_ROWS = N * ROWS_PER_SAMPLE
CHUNK = 128
NCHUNKS = ROWS_PER_SAMPLE // CHUNK
NUM_WORKERS = 32


def _sc_body(img_hbm, scores_hbm, out_hbm,
             scores_v, perm_v, idx_v, buf_v, sem):
    wid = lax.axis_index("s") * 2 + lax.axis_index("c")
    base = wid * ROWS_PER_SAMPLE

    pltpu.sync_copy(scores_hbm.at[wid], scores_v)
    lanes = lax.iota(jnp.int32, 16)
    s = scores_v[...]
    rank = lanes & 0
    for t in range(16):
        st = s[t]
        cond = (st < s) | ((st == s) & (t < lanes))
        rank = rank + jnp.where(cond, 1, 0)
    plsc.store_scatter(perm_v, [rank], lanes)

    def chunk_body(k, carry):
        for u in range(CHUNK // 16):
            j = k * CHUNK + u * 16 + lanes
            cb = j & 3
            y = (j >> 2) & (H - 1)
            c = j >> 11
            i = y & (PH - 1)
            r = y >> 7
            p = (r << 2) | cb
            q = plsc.load_gather(perm_v, [p])
            qr = q >> 2
            qc = q & 3
            row = base + (((c << 9) + (qr << 7) + i) << 2) + qc
            idx_v[pl.ds(u * 16, 16)] = row
        pltpu.async_copy(img_hbm.at[idx_v], buf_v, sem).wait()
        pltpu.sync_copy(buf_v, out_hbm.at[pl.ds(base + k * CHUNK, CHUNK)])
        return carry

    lax.fori_loop(0, NCHUNKS, chunk_body, 0)


@jax.jit
def kernel(img):
    assert img.shape == (N, C, H, W)
    pkey = jax.random.key(42)
    scores = jax.random.uniform(pkey, (N, GRID * GRID), dtype=jnp.float32)

    img2d = img.reshape(TOTAL_ROWS, PW)
    mesh = plsc.VectorSubcoreMesh(core_axis_name="c", subcore_axis_name="s")
    run = functools.partial(
        pl.kernel,
        mesh=mesh,
        out_type=jax.ShapeDtypeStruct((TOTAL_ROWS, PW), jnp.float32),
        scratch_types=[
            pltpu.VMEM((16,), jnp.float32),
            pltpu.VMEM((16,), jnp.int32),
            pltpu.VMEM((CHUNK,), jnp.int32),
            pltpu.VMEM((CHUNK, PW), jnp.float32),
            pltpu.SemaphoreType.DMA,
        ],
        compiler_params=pltpu.CompilerParams(needs_layout_passes=False),
    )(_sc_body)
    out2d = run(img2d, scores)
    return out2d.reshape(N, C, H, W)


# speedup vs baseline: 1.7487x; 1.1153x over previous
import functools

import jax
import jax.numpy as jnp
from jax import lax
from jax.experimental import pallas as pl
from jax.experimental.pallas import tpu as pltpu
from jax.experimental.pallas import tpu_sc as plsc

N, C, H, W = 32, 3, 512, 512
GRID = 4
PH, PW = H // GRID, W // GRID
ROWS_PER_SAMPLE = C * H * (W // PW)
TOTAL_ROWS = N * ROWS_PER_SAMPLE
CHUNK = 128
NCHUNKS = ROWS_PER_SAMPLE // CHUNK
NBUF = 4
NROUNDS = NCHUNKS // NBUF


def _sc_body(img_hbm, scores_hbm, out_hbm,
             scores_v, perm_v, idx_v, buf_v, gsem, wsem):
    wid = lax.axis_index("s") * 2 + lax.axis_index("c")
    base = wid * ROWS_PER_SAMPLE
    lanes = lax.iota(jnp.int32, 16)

    pltpu.sync_copy(scores_hbm.at[wid], scores_v)
    s = scores_v[...]
    rank = lanes & 0
    for t in range(16):
        st = s[t]
        cond = (st < s) | ((st == s) & (t < lanes))
        rank = rank + jnp.where(cond, 1, 0)
    plsc.store_scatter(perm_v, [rank], lanes)

    def fill_idx(k, b):
        for u in range(CHUNK // 16):
            j = k * CHUNK + u * 16 + lanes
            cb = j & 3
            y = (j >> 2) & (H - 1)
            c = j >> 11
            i = y & (PH - 1)
            r = y >> 7
            p = (r << 2) | cb
            q = plsc.load_gather(perm_v, [p])
            qr = q >> 2
            qc = q & 3
            row = base + (((c << 9) + (qr << 7) + i) << 2) + qc
            idx_v.at[b][pl.ds(u * 16, 16)] = row

    def gather(b):
        return pltpu.make_async_copy(img_hbm.at[idx_v.at[b]], buf_v.at[b],
                                     gsem.at[b])

    def writeback(k, b):
        return pltpu.make_async_copy(
            buf_v.at[b], out_hbm.at[pl.ds(base + k * CHUNK, CHUNK)],
            wsem.at[b])

    for b in range(NBUF):
        fill_idx(b, b)
        gather(b).start()

    def round_body(k2, carry):
        k0 = k2 * NBUF
        for b in range(NBUF):
            gather(b).wait()
            writeback(k0 + b, b).start()
        @pl.when(k2 < NROUNDS - 1)
        def _():
            for b in range(NBUF):
                fill_idx(k0 + NBUF + b, b)
                writeback(k0 + b, b).wait()
                gather(b).start()
        return carry

    lax.fori_loop(0, NROUNDS, round_body, 0)

    for b in range(NBUF):
        writeback(NCHUNKS - NBUF + b, b).wait()


@jax.jit
def kernel(img):
    assert img.shape == (N, C, H, W)
    pkey = jax.random.key(42)
    scores = jax.random.uniform(pkey, (N, GRID * GRID), dtype=jnp.float32)

    img2d = img.reshape(TOTAL_ROWS, PW)
    mesh = plsc.VectorSubcoreMesh(core_axis_name="c", subcore_axis_name="s")
    run = functools.partial(
        pl.kernel,
        mesh=mesh,
        out_type=jax.ShapeDtypeStruct((TOTAL_ROWS, PW), jnp.float32),
        scratch_types=[
            pltpu.VMEM((16,), jnp.float32),
            pltpu.VMEM((16,), jnp.int32),
            pltpu.VMEM((NBUF, CHUNK), jnp.int32),
            pltpu.VMEM((NBUF, CHUNK, PW), jnp.float32),
            pltpu.SemaphoreType.DMA((NBUF,)),
            pltpu.SemaphoreType.DMA((NBUF,)),
        ],
        compiler_params=pltpu.CompilerParams(needs_layout_passes=False),
    )(_sc_body)
    out2d = run(img2d, scores)
    return out2d.reshape(N, C, H, W)


# speedup vs baseline: 5.4805x; 3.1340x over previous
import functools

import jax
import jax.numpy as jnp
from jax import lax
from jax.experimental import pallas as pl
from jax.experimental.pallas import tpu as pltpu
from jax.experimental.pallas import tpu_sc as plsc

N, C, H, W = 32, 3, 512, 512
GRID = 4
PH, PW = H // GRID, W // GRID
NCHUNKS = C * GRID * GRID
NBUF = 4
NROUNDS = NCHUNKS // NBUF


def _sc_body(img_hbm, scores_hbm, out_hbm, scores_v, buf_v, gsem, wsem):
    wid = lax.axis_index("s") * 2 + lax.axis_index("c")
    lanes = lax.iota(jnp.int32, 16)

    pltpu.sync_copy(scores_hbm.at[wid], scores_v)
    s = scores_v[...]
    rank = lanes & 0
    for t in range(16):
        st = s[t]
        cond = (st < s) | ((st == s) & (t < lanes))
        rank = rank + jnp.where(cond, 1, 0)

    def gather(k, b):
        c = k >> 4
        p = k & 15
        q = jnp.sum(jnp.where(rank == p, lanes, 0))
        qr = q >> 2
        qc = q & 3
        return pltpu.make_async_copy(
            img_hbm.at[wid, c, pl.ds(qr * PH, PH), pl.ds(qc * PW, PW)],
            buf_v.at[b], gsem.at[b])

    def writeback(k, b):
        c = k >> 4
        p = k & 15
        r = p >> 2
        cb = p & 3
        return pltpu.make_async_copy(
            buf_v.at[b],
            out_hbm.at[wid, c, pl.ds(r * PH, PH), pl.ds(cb * PW, PW)],
            wsem.at[b])

    for b in range(NBUF):
        gather(b, b).start()

    def round_body(k2, carry):
        k0 = k2 * NBUF
        for b in range(NBUF):
            gather(k0 + b, b).wait()
            writeback(k0 + b, b).start()
        @pl.when(k2 < NROUNDS - 1)
        def _():
            for b in range(NBUF):
                writeback(k0 + b, b).wait()
                gather(k0 + NBUF + b, b).start()
        return carry

    lax.fori_loop(0, NROUNDS, round_body, 0)

    for b in range(NBUF):
        writeback(NCHUNKS - NBUF + b, b).wait()


@jax.jit
def kernel(img):
    assert img.shape == (N, C, H, W)
    pkey = jax.random.key(42)
    scores = jax.random.uniform(pkey, (N, GRID * GRID), dtype=jnp.float32)

    mesh = plsc.VectorSubcoreMesh(core_axis_name="c", subcore_axis_name="s")
    run = functools.partial(
        pl.kernel,
        mesh=mesh,
        out_type=jax.ShapeDtypeStruct((N, C, H, W), jnp.float32),
        scratch_types=[
            pltpu.VMEM((16,), jnp.float32),
            pltpu.VMEM((NBUF, PH, PW), jnp.float32),
            pltpu.SemaphoreType.DMA((NBUF,)),
            pltpu.SemaphoreType.DMA((NBUF,)),
        ],
        compiler_params=pltpu.CompilerParams(needs_layout_passes=False),
    )(_sc_body)
    return run(img, scores)
